# baseline (device time: 134224 ns/iter reference)
import functools

import jax
import jax.numpy as jnp
from jax import lax
from jax.experimental import pallas as pl
from jax.experimental.pallas import tpu as pltpu

T = 2048
D = 4096
V_LOCAL = 8192
T_LOC = T // 2
V_CHUNK = 256
N_CHUNKS = V_LOCAL // V_CHUNK


def _body(x_ref, w_ref, lab_ref, out_ref,
          stats, peer_y, nll, peer_x,
          send_y, recv_y, send_x, recv_x):
    k = pl.program_id(0)
    my_x = lax.axis_index("x")
    my_y = lax.axis_index("y")

    @pl.when(k == 0)
    def _init():
        bar = pltpu.get_barrier_semaphore()
        pl.semaphore_signal(bar, inc=1, device_id=(my_x, 1 - my_y),
                            device_id_type=pl.DeviceIdType.MESH)
        pl.semaphore_signal(bar, inc=1, device_id=(1 - my_x, my_y),
                            device_id_type=pl.DeviceIdType.MESH)
        pl.semaphore_wait(bar, 2)
        stats[0, :, :] = jnp.zeros((T_LOC, 1), jnp.float32)
        stats[1, :, :] = jnp.zeros((T_LOC, 1), jnp.float32)

    logits = jnp.dot(x_ref[...], w_ref[...],
                     preferred_element_type=jnp.float32)
    col0 = my_y * V_LOCAL + k * V_CHUNK
    cols = col0 + lax.broadcasted_iota(jnp.int32, (T_LOC, V_CHUNK), 1)
    hit = jnp.where(cols == lab_ref[...], logits, 0.0)
    stats[0, :, :] = stats[0, :, :] + jnp.sum(
        jnp.exp(logits), axis=1, keepdims=True)
    stats[1, :, :] = stats[1, :, :] + jnp.sum(hit, axis=1, keepdims=True)

    @pl.when(k == N_CHUNKS - 1)
    def _finish():
        rdma_y = pltpu.make_async_remote_copy(
            src_ref=stats, dst_ref=peer_y,
            send_sem=send_y, recv_sem=recv_y,
            device_id=(my_x, 1 - my_y),
            device_id_type=pl.DeviceIdType.MESH,
        )
        rdma_y.start()
        rdma_y.wait()

        s_g = stats[0, :, :] + peer_y[0, :, :]
        l_g = stats[1, :, :] + peer_y[1, :, :]
        nll[...] = jnp.log(s_g) - l_g

        rdma_x = pltpu.make_async_remote_copy(
            src_ref=nll, dst_ref=peer_x,
            send_sem=send_x, recv_sem=recv_x,
            device_id=(1 - my_x, my_y),
            device_id_type=pl.DeviceIdType.MESH,
        )
        rdma_x.start()
        rdma_x.wait()

        out_ref[pl.ds(my_x * T_LOC, T_LOC), :] = nll[...]
        out_ref[pl.ds((1 - my_x) * T_LOC, T_LOC), :] = peer_x[...]


def kernel(x, W, labels):
    my_x = lax.axis_index("x")
    x_loc = lax.dynamic_slice_in_dim(x, my_x * T_LOC, T_LOC, axis=0)
    lab_loc = lax.dynamic_slice_in_dim(labels, my_x * T_LOC, T_LOC).reshape(
        T_LOC, 1)

    out = pl.pallas_call(
        _body,
        grid=(N_CHUNKS,),
        in_specs=[
            pl.BlockSpec((T_LOC, D), lambda k: (0, 0)),
            pl.BlockSpec((D, V_CHUNK), lambda k: (0, k)),
            pl.BlockSpec((T_LOC, 1), lambda k: (0, 0)),
        ],
        out_specs=pl.BlockSpec((T, 1), lambda k: (0, 0)),
        out_shape=jax.ShapeDtypeStruct((T, 1), jnp.float32),
        scratch_shapes=[
            pltpu.VMEM((2, T_LOC, 1), jnp.float32),
            pltpu.VMEM((2, T_LOC, 1), jnp.float32),
            pltpu.VMEM((T_LOC, 1), jnp.float32),
            pltpu.VMEM((T_LOC, 1), jnp.float32),
            pltpu.SemaphoreType.DMA,
            pltpu.SemaphoreType.DMA,
            pltpu.SemaphoreType.DMA,
            pltpu.SemaphoreType.DMA,
        ],
        compiler_params=pltpu.CompilerParams(
            collective_id=0, vmem_limit_bytes=100 * 1024 * 1024),
    )(x_loc, W, lab_loc)
    return out.reshape(T)


# device time: 115515 ns/iter; 1.1620x vs baseline; 1.1620x over previous
import functools

import jax
import jax.numpy as jnp
from jax import lax
from jax.experimental import pallas as pl
from jax.experimental.pallas import tpu as pltpu

T = 2048
D = 4096
V_LOCAL = 8192
T_LOC = T // 2
V_CHUNK = 512
N_CHUNKS = V_LOCAL // V_CHUNK


def _body(x_ref, w_ref, lab_ref, out_ref,
          stats, peer_y, nll, peer_x,
          send_y, recv_y, send_x, recv_x):
    k = pl.program_id(0)
    my_x = lax.axis_index("x")
    my_y = lax.axis_index("y")

    @pl.when(k == 0)
    def _init():
        bar = pltpu.get_barrier_semaphore()
        pl.semaphore_signal(bar, inc=1, device_id=(my_x, 1 - my_y),
                            device_id_type=pl.DeviceIdType.MESH)
        pl.semaphore_signal(bar, inc=1, device_id=(1 - my_x, my_y),
                            device_id_type=pl.DeviceIdType.MESH)
        pl.semaphore_wait(bar, 2)
        stats[0, :, :] = jnp.zeros((T_LOC, 1), jnp.float32)
        stats[1, :, :] = jnp.zeros((T_LOC, 1), jnp.float32)

    logits = jnp.dot(x_ref[...], w_ref[...],
                     preferred_element_type=jnp.float32)
    col0 = my_y * V_LOCAL + k * V_CHUNK
    cols = col0 + lax.broadcasted_iota(jnp.int32, (T_LOC, V_CHUNK), 1)
    hit = jnp.where(cols == lab_ref[...], logits, 0.0)
    stats[0, :, :] = stats[0, :, :] + jnp.sum(
        jnp.exp(logits), axis=1, keepdims=True)
    stats[1, :, :] = stats[1, :, :] + jnp.sum(hit, axis=1, keepdims=True)

    @pl.when(k == N_CHUNKS - 1)
    def _finish():
        rdma_y = pltpu.make_async_remote_copy(
            src_ref=stats, dst_ref=peer_y,
            send_sem=send_y, recv_sem=recv_y,
            device_id=(my_x, 1 - my_y),
            device_id_type=pl.DeviceIdType.MESH,
        )
        rdma_y.start()
        rdma_y.wait()

        s_g = stats[0, :, :] + peer_y[0, :, :]
        l_g = stats[1, :, :] + peer_y[1, :, :]
        nll[...] = jnp.log(s_g) - l_g

        rdma_x = pltpu.make_async_remote_copy(
            src_ref=nll, dst_ref=peer_x,
            send_sem=send_x, recv_sem=recv_x,
            device_id=(1 - my_x, my_y),
            device_id_type=pl.DeviceIdType.MESH,
        )
        rdma_x.start()
        rdma_x.wait()

        out_ref[pl.ds(my_x * T_LOC, T_LOC), :] = nll[...]
        out_ref[pl.ds((1 - my_x) * T_LOC, T_LOC), :] = peer_x[...]


def kernel(x, W, labels):
    my_x = lax.axis_index("x")
    x_loc = lax.dynamic_slice_in_dim(x, my_x * T_LOC, T_LOC, axis=0)
    lab_loc = lax.dynamic_slice_in_dim(labels, my_x * T_LOC, T_LOC).reshape(
        T_LOC, 1)

    out = pl.pallas_call(
        _body,
        grid=(N_CHUNKS,),
        in_specs=[
            pl.BlockSpec((T_LOC, D), lambda k: (0, 0)),
            pl.BlockSpec((D, V_CHUNK), lambda k: (0, k)),
            pl.BlockSpec((T_LOC, 1), lambda k: (0, 0)),
        ],
        out_specs=pl.BlockSpec((T, 1), lambda k: (0, 0)),
        out_shape=jax.ShapeDtypeStruct((T, 1), jnp.float32),
        scratch_shapes=[
            pltpu.VMEM((2, T_LOC, 1), jnp.float32),
            pltpu.VMEM((2, T_LOC, 1), jnp.float32),
            pltpu.VMEM((T_LOC, 1), jnp.float32),
            pltpu.VMEM((T_LOC, 1), jnp.float32),
            pltpu.SemaphoreType.DMA,
            pltpu.SemaphoreType.DMA,
            pltpu.SemaphoreType.DMA,
            pltpu.SemaphoreType.DMA,
        ],
        compiler_params=pltpu.CompilerParams(collective_id=0),
    )(x_loc, W, lab_loc)
    return out.reshape(T)
